# baseline (speedup 1.0000x reference)
import functools

import jax
import jax.numpy as jnp
from jax import lax
from jax.experimental import pallas as pl
from jax.experimental.pallas import tpu as pltpu
from jax.experimental.pallas import tpu_sc as plsc

NUM_REL = 3
N = 10000
E = 320000
D = 128

NUM_CORES = 2
NUM_SUBCORES = 16
NW = NUM_CORES * NUM_SUBCORES
CH = 128
NCHUNK = 4 * (-(-E // (NW * CH * 4)))
EPAD = NW * CH * NCHUNK
ACC_ROWS = 10240
ROWS_PER_TILE = ACC_ROWS // NUM_SUBCORES
MM_BLK = 2000
NB = N // MM_BLK


def _mm_body(x_ref, w_ref, o_ref):
    o_ref[...] = jnp.dot(x_ref[...], w_ref[0],
                         preferred_element_type=jnp.float32)


def _project(x, w):
    return pl.pallas_call(
        _mm_body,
        grid=(NUM_REL, NB),
        in_specs=[
            pl.BlockSpec((MM_BLK, D), lambda r, n: (n, 0)),
            pl.BlockSpec((1, D, D), lambda r, n: (r, 0, 0)),
        ],
        out_specs=pl.BlockSpec((MM_BLK, D), lambda r, n: (r * NB + n, 0)),
        out_shape=jax.ShapeDtypeStruct((NUM_REL * N, D), jnp.float32),
    )(x, w)


def _relu_mm_body(p_ref, w_ref, o_ref):
    h = jnp.maximum(p_ref[0] + p_ref[1], 0.0)
    o_ref[...] = jnp.dot(h, w_ref[0], preferred_element_type=jnp.float32)


def _relu_project(p, w):
    return pl.pallas_call(
        _relu_mm_body,
        grid=(NUM_REL, NB),
        in_specs=[
            pl.BlockSpec((NUM_CORES, MM_BLK, D), lambda r, n: (0, n, 0)),
            pl.BlockSpec((1, D, D), lambda r, n: (r, 0, 0)),
        ],
        out_specs=pl.BlockSpec((MM_BLK, D), lambda r, n: (r * NB + n, 0)),
        out_shape=jax.ShapeDtypeStruct((NUM_REL * N, D), jnp.float32),
    )(p, w)


def _logsoftmax_body(p_ref, o_ref):
    x = p_ref[0] + p_ref[1]
    m = jnp.max(x, axis=1, keepdims=True)
    ex = jnp.exp(x - m)
    lse = jnp.log(jnp.sum(ex, axis=1, keepdims=True)) + m
    o_ref[...] = x - lse


def _sum_logsoftmax(p):
    return pl.pallas_call(
        _logsoftmax_body,
        grid=(NB,),
        in_specs=[pl.BlockSpec((NUM_CORES, MM_BLK, D), lambda n: (0, n, 0))],
        out_specs=pl.BlockSpec((MM_BLK, D), lambda n: (n, 0)),
        out_shape=jax.ShapeDtypeStruct((N, D), jnp.float32),
    )(p)


PK_BLK = 256


def _pack_body(s_ref, t_ref, d_ref, o_ref):
    o_ref[:, 0, :] = t_ref[...] * N + s_ref[...]
    o_ref[:, 1, :] = d_ref[...]


def _pack_idx(s2, t2, d2):
    k = s2.shape[0]
    return pl.pallas_call(
        _pack_body,
        grid=(k // PK_BLK,),
        in_specs=[pl.BlockSpec((PK_BLK, CH), lambda i: (i, 0))] * 3,
        out_specs=pl.BlockSpec((PK_BLK, 2, CH), lambda i: (i, 0, 0)),
        out_shape=jax.ShapeDtypeStruct((k, 2, CH), jnp.int32),
    )(s2, t2, d2)


_SC_MESH = plsc.VectorSubcoreMesh(core_axis_name="c", subcore_axis_name="s")


NROW = 2
NPK = 4
NGRP = NCHUNK // NPK


@functools.partial(
    pl.kernel,
    out_type=jax.ShapeDtypeStruct((NUM_CORES, ACC_ROWS, D), jnp.float32),
    mesh=_SC_MESH,
    scratch_types=[
        [pltpu.VMEM((2, CH), jnp.int32)] * NPK,
        [pltpu.VMEM((CH, D), jnp.float32)] * NROW,
        pltpu.VMEM_SHARED((ACC_ROWS, D), jnp.float32),
        [pltpu.SemaphoreType.DMA] * NPK,
        [pltpu.SemaphoreType.DMA] * NROW,
        [pltpu.SemaphoreType.DMA] * NROW,
    ],
)
def _sc_aggregate(y_hbm, pk_hbm, zeros_hbm, out_hbm,
                  pk, rows, acc, psem, gsem, ssem):
    c = lax.axis_index("c")
    s = lax.axis_index("s")
    wid = c * NUM_SUBCORES + s
    base = wid * NCHUNK

    def mk_pk_load(j, q):
        return pltpu.make_async_copy(pk_hbm.at[base + j], pk[q], psem[q])

    def mk_gather(j, r, q):
        return pltpu.make_async_copy(y_hbm.at[pk[q].at[0]], rows[r], gsem[r])

    def mk_scatter(r, q):
        return pltpu.make_async_copy(rows[r], acc.at[pk[q].at[1]], ssem[r])

    for j in range(3):
        mk_pk_load(j, j).start()
    pltpu.sync_copy(zeros_hbm, acc.at[pl.ds(s * ROWS_PER_TILE, ROWS_PER_TILE)])
    plsc.subcore_barrier()

    mk_pk_load(0, 0).wait()
    mk_gather(0, 0, 0).start()

    def step(j, g, u, first_group, last_group):
        r, q = u % NROW, u % NPK
        rn, qn = (u + 1) % NROW, (u + 1) % NPK
        if not (first_group and u == 0):
            mk_scatter(rn, (u - 1) % NPK).wait()
        if not (last_group and u == NPK - 1):
            mk_pk_load(g * NPK + u + 1, qn).wait()
            mk_gather(j + 1, rn, qn).start()
        mk_gather(j, r, q).wait()
        mk_scatter(r, q).start(add=True)
        if (not last_group) or (u + 3 < NPK):
            mk_pk_load(g * NPK + u + 3, (u + 3) % NPK).start()

    for u in range(NPK):
        step(u, 0, u, True, False)

    def body(g, carry):
        for u in range(NPK):
            step(g * NPK + u, g, u, False, False)
        return carry

    lax.fori_loop(1, NGRP - 1, body, 0)

    for u in range(NPK):
        step((NGRP - 1) * NPK + u, NGRP - 1, u, False, True)

    mk_scatter((NCHUNK - 1) % NROW, (NCHUNK - 1) % NPK).wait()

    plsc.subcore_barrier()

    sl = pl.ds(s * ROWS_PER_TILE, ROWS_PER_TILE)
    pltpu.sync_copy(acc.at[sl], out_hbm.at[c, sl])


@jax.jit
def kernel(X, edge_index, edge_type, W1, W2):
    pad = EPAD - E
    k = NW * NCHUNK
    src = jnp.concatenate([edge_index[0], jnp.zeros((pad,), jnp.int32)])
    typ = jnp.concatenate([edge_type, jnp.zeros((pad,), jnp.int32)])
    dst = jnp.concatenate([edge_index[1], jnp.full((pad,), N, jnp.int32)])
    pk = _pack_idx(src.reshape(k, CH), typ.reshape(k, CH), dst.reshape(k, CH))
    zeros = jnp.zeros((ROWS_PER_TILE, D), jnp.float32)

    y1 = _project(X, W1)
    p1 = _sc_aggregate(y1, pk, zeros)
    y2 = _relu_project(p1, W2)
    p2 = _sc_aggregate(y2, pk, zeros)
    return _sum_logsoftmax(p2)
